# baseline (speedup 1.0000x reference)
import functools

import jax
import jax.numpy as jnp
from jax import lax
from jax.experimental import pallas as pl
from jax.experimental.pallas import tpu as pltpu
from jax.experimental.pallas import tpu_sc as plsc

_FIELD_DIMS = (32, 32, 16, 32, 32, 64)
_DM = 192
_EPS = 1e-5

_B, _S = 1024, 200
_N = _B * _S
_NC, _NS = 2, 16
_NW = _NC * _NS
_PER_W = _N // _NW
_CHUNK = 128
_N_CHUNKS = _PER_W // _CHUNK

_PACK = (
    (0, 0, 0),
    (1, 0, 32),
    (5, 0, 64),
    (2, 1, 0),
    (3, 1, 16),
    (4, 1, 48),
    (3, 1, 80),
    (2, 1, 112),
)


def _sc_gather(ids2d, tabs):
    mesh = plsc.VectorSubcoreMesh(core_axis_name="c", subcore_axis_name="s")
    scratch = []
    for _ in range(6):
        scratch.append(pltpu.VMEM((_N_CHUNKS, _CHUNK), jnp.int32))
    for _ in range(2):
        for d in _FIELD_DIMS:
            scratch.append(pltpu.VMEM((_CHUNK, d), jnp.float32))
    scratch.append(pltpu.SemaphoreType.DMA)
    scratch.append(pltpu.SemaphoreType.DMA)
    scratch.append(pltpu.SemaphoreType.DMA)
    scratch.append(pltpu.SemaphoreType.DMA)
    scratch.append(pltpu.SemaphoreType.DMA)

    @functools.partial(
        pl.kernel,
        mesh=mesh,
        out_type=(jax.ShapeDtypeStruct((_N, 128), jnp.float32),
                  jax.ShapeDtypeStruct((_N, 128), jnp.float32)),
        scratch_types=scratch,
        compiler_params=pltpu.CompilerParams(use_tc_tiling_on_sc=False),
    )
    def k(i0, i1, i2, i3, i4, i5, t0, t1, t2, t3, t4, t5, oa, ob,
          x0, x1, x2, x3, x4, x5,
          r00, r01, r02, r03, r04, r05,
          r10, r11, r12, r13, r14, r15,
          sem_l, sg0, sg1, sw0, sw1):
        ids_refs = (i0, i1, i2, i3, i4, i5)
        idx_refs = (x0, x1, x2, x3, x4, x5)
        src_refs = (t0, t1, t2, t3, t4, t5)
        rows = ((r00, r01, r02, r03, r04, r05),
                (r10, r11, r12, r13, r14, r15))
        out_refs = (oa, ob)
        sems_g = (sg0, sg1)
        sems_w = (sw0, sw1)
        wid = lax.axis_index("s") * _NC + lax.axis_index("c")
        base0 = wid * _PER_W

        idx_cps = [
            pltpu.async_copy(
                ids_refs[f].at[pl.ds(wid * _N_CHUNKS, _N_CHUNKS)],
                idx_refs[f], sem_l)
            for f in range(6)
        ]
        for cp in idx_cps:
            cp.wait()

        def issue_gathers(c, buf):
            for f in range(6):
                pltpu.async_copy(src_refs[f].at[idx_refs[f].at[c]],
                                 rows[buf][f], sems_g[buf])

        def wait_gathers(c, buf):
            for f in range(6):
                pltpu.make_async_copy(src_refs[f].at[idx_refs[f].at[c]],
                                      rows[buf][f], sems_g[buf]).wait()

        def write_descs(base, buf, make_only):
            for f, o, col in _PACK:
                dst = out_refs[o].at[pl.ds(base, _CHUNK),
                                     pl.ds(col, _FIELD_DIMS[f])]
                if make_only:
                    pltpu.make_async_copy(rows[buf][f], dst,
                                          sems_w[buf]).wait()
                else:
                    pltpu.async_copy(rows[buf][f], dst, sems_w[buf])

        issue_gathers(0, 0)
        issue_gathers(1, 1)

        def super_step(s, carry):
            for buf in range(2):
                c = 2 * s + buf
                base = pl.multiple_of(base0 + c * _CHUNK, _CHUNK)
                wait_gathers(c, buf)
                write_descs(base, buf, make_only=False)

                @pl.when(s < (_N_CHUNKS // 2) - 1)
                def _():
                    write_descs(base, buf, make_only=True)
                    issue_gathers(c + 2, buf)
            return carry

        lax.fori_loop(0, _N_CHUNKS // 2, super_step, 0)

        last0 = pl.multiple_of(base0 + (_N_CHUNKS - 2) * _CHUNK, _CHUNK)
        last1 = pl.multiple_of(base0 + (_N_CHUNKS - 1) * _CHUNK, _CHUNK)
        write_descs(last0, 0, make_only=True)
        write_descs(last1, 1, make_only=True)

    return k(*ids2d, *tabs)


def _tc_project_ln(a, bb, wa, wb, b2, g2, be2):
    def body(a_ref, b_ref, wa_ref, wb_ref, bias_ref, g_ref, be_ref, o_ref):
        y = jnp.dot(a_ref[...], wa_ref[...],
                    preferred_element_type=jnp.float32)
        y = y + jnp.dot(b_ref[...], wb_ref[...],
                        preferred_element_type=jnp.float32)
        y = y + bias_ref[...]
        mu = jnp.mean(y, axis=-1, keepdims=True)
        d = y - mu
        var = jnp.mean(d * d, axis=-1, keepdims=True)
        z = d * lax.rsqrt(var + _EPS) * g_ref[...] + be_ref[...]
        o_ref[0] = z.T

    return pl.pallas_call(
        body,
        grid=(_S,),
        in_specs=[
            pl.BlockSpec((_B, 128), lambda i: (i, 0)),
            pl.BlockSpec((_B, 128), lambda i: (i, 0)),
            pl.BlockSpec((128, _DM), lambda i: (0, 0)),
            pl.BlockSpec((128, _DM), lambda i: (0, 0)),
            pl.BlockSpec((1, _DM), lambda i: (0, 0)),
            pl.BlockSpec((1, _DM), lambda i: (0, 0)),
            pl.BlockSpec((1, _DM), lambda i: (0, 0)),
        ],
        out_specs=pl.BlockSpec((1, _DM, _B), lambda i: (i, 0, 0)),
        out_shape=jax.ShapeDtypeStruct((_S, _DM, _B), jnp.float32),
    )(a, bb, wa, wb, b2, g2, be2)


def kernel(input_ids_pos, input_ids_pos_detail1, input_ids_pos_detail2,
           input_ids_conjugated_type, input_ids_conjugated_form,
           input_ids_lemma,
           table_pos, table_pos_detail1, table_pos_detail2,
           table_conjugated_type, table_conjugated_form, table_lemma,
           W, b, gamma, beta):
    ids2d = [a.T.reshape(_N // _CHUNK, _CHUNK) for a in (
        input_ids_pos, input_ids_pos_detail1, input_ids_pos_detail2,
        input_ids_conjugated_type, input_ids_conjugated_form,
        input_ids_lemma)]
    small = [table_pos, table_pos_detail1, table_pos_detail2,
             table_conjugated_type, table_conjugated_form]
    t2 = table_lemma.reshape(500000, 128)
    t2 = jax.lax.optimization_barrier(t2)
    tl = t2.reshape(1000000, 64)
    tabs = [jnp.tile(t, (_NW, 1)) for t in small] + [tl]
    wvec = jnp.arange(_N // _CHUNK, dtype=jnp.int32) // _N_CHUNKS
    for f in range(5):
        ids2d[f] = ids2d[f] + (wvec * small[f].shape[0])[:, None]
    a, bb = _sc_gather(ids2d, tabs)
    a, bb = jax.lax.optimization_barrier((a.reshape(-1), bb.reshape(-1)))
    a = a.reshape(_N, 128)
    bb = bb.reshape(_N, 128)
    wa = jnp.concatenate([W[0:32], W[32:64], W[144:208]], axis=0)
    wb = jnp.concatenate([W[64:80], W[80:112], W[112:144],
                          jnp.zeros((48, _DM), jnp.float32)], axis=0)
    out = _tc_project_ln(a, bb, wa, wb, b.reshape(1, _DM),
                         gamma.reshape(1, _DM), beta.reshape(1, _DM))
    return out.transpose(2, 0, 1)


# speedup vs baseline: 1.0149x; 1.0149x over previous
import functools

import jax
import jax.numpy as jnp
from jax import lax
from jax.experimental import pallas as pl
from jax.experimental.pallas import tpu as pltpu
from jax.experimental.pallas import tpu_sc as plsc

_FIELD_DIMS = (32, 32, 16, 32, 32, 64)
_DM = 192
_EPS = 1e-5

_B, _S = 1024, 200
_N = _B * _S
_NC, _NS = 2, 16
_NW = _NC * _NS
_PER_W = _N // _NW
_CHUNK = 128
_N_CHUNKS = _PER_W // _CHUNK

def _sc_gather_generic(ids2d_sel, tabs_sel, pack):
    nf = len(tabs_sel)
    dims = [t.shape[1] for t in tabs_sel]
    mesh = plsc.VectorSubcoreMesh(core_axis_name="c", subcore_axis_name="s")
    scratch = []
    for _ in range(nf):
        scratch.append(pltpu.VMEM((_N_CHUNKS, _CHUNK), jnp.int32))
    for _ in range(2):
        for d in dims:
            scratch.append(pltpu.VMEM((_CHUNK, d), jnp.float32))
    for _ in range(5):
        scratch.append(pltpu.SemaphoreType.DMA)

    @functools.partial(
        pl.kernel,
        mesh=mesh,
        out_type=jax.ShapeDtypeStruct((_N, 128), jnp.float32),
        scratch_types=scratch,
        compiler_params=pltpu.CompilerParams(use_tc_tiling_on_sc=False),
    )
    def k(*refs):
        ids_refs = refs[:nf]
        src_refs = refs[nf:2 * nf]
        out = refs[2 * nf]
        sc = refs[2 * nf + 1:]
        idx_refs = sc[:nf]
        rows = (sc[nf:2 * nf], sc[2 * nf:3 * nf])
        sem_l, sg0, sg1, sw0, sw1 = sc[3 * nf:]
        sems_g = (sg0, sg1)
        sems_w = (sw0, sw1)
        wid = lax.axis_index("s") * _NC + lax.axis_index("c")
        base0 = wid * _PER_W

        idx_cps = [
            pltpu.async_copy(
                ids_refs[f].at[pl.ds(wid * _N_CHUNKS, _N_CHUNKS)],
                idx_refs[f], sem_l)
            for f in range(nf)
        ]
        for cp in idx_cps:
            cp.wait()

        def issue_gathers(c, buf):
            for f in range(nf):
                pltpu.async_copy(src_refs[f].at[idx_refs[f].at[c]],
                                 rows[buf][f], sems_g[buf])

        def wait_gathers(c, buf):
            for f in range(nf):
                pltpu.make_async_copy(src_refs[f].at[idx_refs[f].at[c]],
                                      rows[buf][f], sems_g[buf]).wait()

        def write_descs(base, buf, make_only):
            for f, col in pack:
                dst = out.at[pl.ds(base, _CHUNK), pl.ds(col, dims[f])]
                if make_only:
                    pltpu.make_async_copy(rows[buf][f], dst,
                                          sems_w[buf]).wait()
                else:
                    pltpu.async_copy(rows[buf][f], dst, sems_w[buf])

        issue_gathers(0, 0)
        issue_gathers(1, 1)

        def super_step(s, carry):
            for buf in range(2):
                c = 2 * s + buf
                base = pl.multiple_of(base0 + c * _CHUNK, _CHUNK)
                wait_gathers(c, buf)
                write_descs(base, buf, make_only=False)

                @pl.when(s < (_N_CHUNKS // 2) - 1)
                def _():
                    write_descs(base, buf, make_only=True)
                    issue_gathers(c + 2, buf)
            return carry

        lax.fori_loop(0, _N_CHUNKS // 2, super_step, 0)

        last0 = pl.multiple_of(base0 + (_N_CHUNKS - 2) * _CHUNK, _CHUNK)
        last1 = pl.multiple_of(base0 + (_N_CHUNKS - 1) * _CHUNK, _CHUNK)
        write_descs(last0, 0, make_only=True)
        write_descs(last1, 1, make_only=True)

    return k(*ids2d_sel, *tabs_sel)


def _tc_project_ln(a, bb, wa, wb, b2, g2, be2):
    def body(a_ref, b_ref, wa_ref, wb_ref, bias_ref, g_ref, be_ref, o_ref):
        y = jnp.dot(a_ref[...], wa_ref[...],
                    preferred_element_type=jnp.float32)
        y = y + jnp.dot(b_ref[...], wb_ref[...],
                        preferred_element_type=jnp.float32)
        y = y + bias_ref[...]
        mu = jnp.mean(y, axis=-1, keepdims=True)
        d = y - mu
        var = jnp.mean(d * d, axis=-1, keepdims=True)
        z = d * lax.rsqrt(var + _EPS) * g_ref[...] + be_ref[...]
        o_ref[0] = z.T

    return pl.pallas_call(
        body,
        grid=(_S,),
        in_specs=[
            pl.BlockSpec((_B, 128), lambda i: (i, 0)),
            pl.BlockSpec((_B, 128), lambda i: (i, 0)),
            pl.BlockSpec((128, _DM), lambda i: (0, 0)),
            pl.BlockSpec((128, _DM), lambda i: (0, 0)),
            pl.BlockSpec((1, _DM), lambda i: (0, 0)),
            pl.BlockSpec((1, _DM), lambda i: (0, 0)),
            pl.BlockSpec((1, _DM), lambda i: (0, 0)),
        ],
        out_specs=pl.BlockSpec((1, _DM, _B), lambda i: (i, 0, 0)),
        out_shape=jax.ShapeDtypeStruct((_S, _DM, _B), jnp.float32),
    )(a, bb, wa, wb, b2, g2, be2)


def kernel(input_ids_pos, input_ids_pos_detail1, input_ids_pos_detail2,
           input_ids_conjugated_type, input_ids_conjugated_form,
           input_ids_lemma,
           table_pos, table_pos_detail1, table_pos_detail2,
           table_conjugated_type, table_conjugated_form, table_lemma,
           W, b, gamma, beta):
    ids2d = [a.T.reshape(_N // _CHUNK, _CHUNK) for a in (
        input_ids_pos, input_ids_pos_detail1, input_ids_pos_detail2,
        input_ids_conjugated_type, input_ids_conjugated_form,
        input_ids_lemma)]
    small = [table_pos, table_pos_detail1, table_pos_detail2,
             table_conjugated_type, table_conjugated_form]
    tabs = [jnp.tile(t, (_NW, 1)) for t in small]
    wvec = jnp.arange(_N // _CHUNK, dtype=jnp.int32) // _N_CHUNKS
    for f in range(5):
        ids2d[f] = ids2d[f] + (wvec * small[f].shape[0])[:, None]
    t2 = table_lemma.reshape(500000, 128)
    t2 = jax.lax.optimization_barrier(t2)
    tl = t2.reshape(1000000, 64)
    bb = _sc_gather_generic(
        [ids2d[0], ids2d[1], ids2d[3], ids2d[4]],
        [tabs[0], tabs[1], tabs[3], tabs[4]],
        ((0, 0), (1, 32), (2, 64), (3, 96)))
    a = _sc_gather_generic(
        [ids2d[2], ids2d[5]],
        [tabs[2], tl],
        ((0, 0), (1, 16), (0, 80), (0, 96), (0, 112)))
    a, bb = jax.lax.optimization_barrier((a.reshape(-1), bb.reshape(-1)))
    a = a.reshape(_N, 128)
    bb = bb.reshape(_N, 128)
    wa = jnp.concatenate([W[64:80], W[144:208],
                          jnp.zeros((48, _DM), jnp.float32)], axis=0)
    wb = jnp.concatenate([W[0:32], W[32:64], W[80:112], W[112:144]], axis=0)
    out = _tc_project_ln(a, bb, wa, wb, b.reshape(1, _DM),
                         gamma.reshape(1, _DM), beta.reshape(1, _DM))
    return out.transpose(2, 0, 1)
